# baseline (speedup 1.0000x reference)
import functools

import jax
import jax.numpy as jnp
from jax import lax
from jax.experimental import pallas as pl
from jax.experimental.pallas import tpu as pltpu
from jax.experimental.pallas import tpu_sc as plsc

_LANES = 16
_CHUNK = 128


def _make_kernel(B, D):
    info = plsc.get_sparse_core_info()
    nc, ns = info.num_cores, info.num_subcores
    nw = nc * ns
    bpw = B // nw
    nch = bpw // _CHUNK
    nd = D // _LANES

    mesh = plsc.VectorSubcoreMesh(core_axis_name="c", subcore_axis_name="s")

    @functools.partial(
        pl.kernel,
        mesh=mesh,
        compiler_params=pltpu.CompilerParams(
            needs_layout_passes=False, use_tc_tiling_on_sc=False),
        out_type=jax.ShapeDtypeStruct((B,), jnp.float32),
        scratch_types=[
            pltpu.VMEM((bpw,), jnp.int32),
            pltpu.VMEM((bpw,), jnp.int32),
            pltpu.VMEM((bpw, D), jnp.float32),
            pltpu.VMEM((bpw, D), jnp.float32),
            pltpu.VMEM((bpw,), jnp.float32),
            pltpu.SemaphoreType.DMA,
            pltpu.SemaphoreType.DMA,
        ],
    )
    def run(users_hbm, items_hbm, uemb_hbm, iemb_hbm, out_hbm,
            uidx, iidx, urows, irows, outv, usem, isem):
        wid = lax.axis_index("s") * nc + lax.axis_index("c")
        base = wid * bpw

        pltpu.sync_copy(users_hbm.at[pl.ds(base, bpw)], uidx)
        pltpu.sync_copy(items_hbm.at[pl.ds(base, bpw)], iidx)

        copies = []
        for j in range(nch):
            sl = pl.ds(j * _CHUNK, _CHUNK)
            copies.append(pltpu.async_copy(
                uemb_hbm.at[uidx.at[sl]], urows.at[sl], usem))
            copies.append(pltpu.async_copy(
                iemb_hbm.at[iidx.at[sl]], irows.at[sl], isem))
        for c in copies:
            c.wait()

        lanes = lax.iota(jnp.int32, _LANES)

        def group(g, carry):
            gb = g * _LANES
            tot = jnp.zeros((_LANES,), jnp.float32)
            for b16 in range(_LANES):
                b = gb + b16
                s = urows[b, pl.ds(0, _LANES)] * irows[b, pl.ds(0, _LANES)]
                for c in range(1, nd):
                    s = s + (urows[b, pl.ds(c * _LANES, _LANES)]
                             * irows[b, pl.ds(c * _LANES, _LANES)])
                tot = jnp.where(lanes == b16, jnp.sum(s), tot)
            outv[pl.ds(gb, _LANES)] = tot
            return carry

        lax.fori_loop(0, bpw // _LANES, group, 0)

        pltpu.sync_copy(outv, out_hbm.at[pl.ds(base, bpw)])

    return run


def kernel(users, items, user_emb, item_emb):
    B = users.shape[0]
    D = user_emb.shape[1]
    users = users.astype(jnp.int32)
    items = items.astype(jnp.int32)
    run = _make_kernel(B, D)
    return run(users, items, user_emb, item_emb)


# speedup vs baseline: 1.5370x; 1.5370x over previous
import functools

import jax
import jax.numpy as jnp
from jax import lax
from jax.experimental import pallas as pl
from jax.experimental.pallas import tpu as pltpu
from jax.experimental.pallas import tpu_sc as plsc

_LANES = 16
_CHUNK = 16


def _make_kernel(B, D):
    info = plsc.get_sparse_core_info()
    nc, ns = info.num_cores, info.num_subcores
    nw = nc * ns
    bpw = B // nw
    nd = D // _LANES
    nch = bpw // _CHUNK

    mesh = plsc.VectorSubcoreMesh(core_axis_name="c", subcore_axis_name="s")

    @functools.partial(
        pl.kernel,
        mesh=mesh,
        compiler_params=pltpu.CompilerParams(needs_layout_passes=False),
        out_type=jax.ShapeDtypeStruct((B,), jnp.float32),
        scratch_types=[
            pltpu.VMEM((bpw,), jnp.int32),
            pltpu.VMEM((bpw,), jnp.int32),
            pltpu.VMEM((_CHUNK, D), jnp.float32),
            pltpu.VMEM((_CHUNK, D), jnp.float32),
            pltpu.VMEM((bpw,), jnp.float32),
            pltpu.SemaphoreType.DMA,
            pltpu.SemaphoreType.DMA,
        ],
    )
    def run(users_hbm, items_hbm, uemb_hbm, iemb_hbm, out_hbm,
            uidx, iidx, ubuf, ibuf, outv, usem, isem):
        wid = lax.axis_index("s") * nc + lax.axis_index("c")
        base = wid * bpw

        pltpu.sync_copy(users_hbm.at[pl.ds(base, bpw)], uidx)
        pltpu.sync_copy(items_hbm.at[pl.ds(base, bpw)], iidx)

        lanes = lax.iota(jnp.int32, _LANES)

        def chunk(j, carry):
            cb = j * _CHUNK
            copies = []
            for g16 in range(_CHUNK // _LANES):
                uv = uidx[pl.ds(cb + g16 * _LANES, _LANES)]
                iv = iidx[pl.ds(cb + g16 * _LANES, _LANES)]
                for k in range(_LANES):
                    b = g16 * _LANES + k
                    copies.append(pltpu.async_copy(
                        uemb_hbm.at[pl.ds(uv[k], 1)],
                        ubuf.at[pl.ds(b, 1)], usem))
                    copies.append(pltpu.async_copy(
                        iemb_hbm.at[pl.ds(iv[k], 1)],
                        ibuf.at[pl.ds(b, 1)], isem))
            for c in copies:
                c.wait()

            for g16 in range(_CHUNK // _LANES):
                gb = g16 * _LANES
                tot = jnp.zeros((_LANES,), jnp.float32)
                for b16 in range(_LANES):
                    k = gb + b16
                    s = ubuf[k, pl.ds(0, _LANES)] * ibuf[k, pl.ds(0, _LANES)]
                    for c in range(1, nd):
                        s = s + (ubuf[k, pl.ds(c * _LANES, _LANES)]
                                 * ibuf[k, pl.ds(c * _LANES, _LANES)])
                    tot = jnp.where(lanes == b16, jnp.sum(s), tot)
                outv[pl.ds(cb + gb, _LANES)] = tot
            return carry

        lax.fori_loop(0, nch, chunk, 0)

        pltpu.sync_copy(outv, out_hbm.at[pl.ds(base, bpw)])

    return run


def kernel(users, items, user_emb, item_emb):
    B = users.shape[0]
    D = user_emb.shape[1]
    users = users.astype(jnp.int32)
    items = items.astype(jnp.int32)
    run = _make_kernel(B, D)
    return run(users, items, user_emb, item_emb)
